# baseline (speedup 1.0000x reference)
import jax
import jax.numpy as jnp
from jax import lax
from jax.experimental import pallas as pl
from jax.experimental.pallas import tpu as pltpu
from jax.experimental.pallas import tpu_sc as plsc

N = 100000
D = 64
B = 16384
ENTITY_AWARE_COFF = 0.001
EWC_LAMBDA = 100.0

NC = 2
NS = 16
LANES = 16
NW = NC * NS
BPW = B // NW
CH = 128
NCHUNK = BPW // CH


def _sc_body(x_hbm, pnpt_hbm, cf_hbm, it_hbm, us_hbm,
             idx_v, u_rows, p_rows, e1_rows, e2_rows,
             cf_b, it_b, us_b, sem0, sem1, sem2):
    wid = lax.axis_index("s") * NC + lax.axis_index("c")
    base = wid * BPW

    pltpu.sync_copy(pnpt_hbm.at[:, wid], idx_v)

    def chunk(c, _):
        off = c * CH
        cp0 = pltpu.async_copy(x_hbm.at[idx_v.at[0, c]], u_rows, sem0)
        cp1 = pltpu.async_copy(x_hbm.at[idx_v.at[1, c]], p_rows, sem1)
        cp2 = pltpu.async_copy(x_hbm.at[idx_v.at[2, c]], e1_rows, sem2)
        cp0.wait()
        cp1.wait()
        cp2.wait()

        def cf_pair(p, _):
            acc = jnp.zeros((LANES,), jnp.float32)
            for d in range(D // LANES):
                ds_ = pl.ds(d * LANES, LANES)
                acc += u_rows[p, ds_] * (p_rows[p, ds_] - e1_rows[p, ds_])
            cf_b[off + p, :] = acc
            return 0

        lax.fori_loop(0, CH, cf_pair, 0, unroll=2)

        cp3 = pltpu.async_copy(x_hbm.at[idx_v.at[3, c]], e1_rows, sem0)
        cp4 = pltpu.async_copy(x_hbm.at[idx_v.at[4, c]], e2_rows, sem1)
        cp3.wait()
        cp4.wait()

        def it_pair(p, _):
            acc = jnp.zeros((LANES,), jnp.float32)
            for d in range(D // LANES):
                ds_ = pl.ds(d * LANES, LANES)
                xi = p_rows[p, ds_]
                dp = xi - e1_rows[p, ds_]
                dn = xi - e2_rows[p, ds_]
                acc += dp * dp - dn * dn
            it_b[off + p, :] = acc
            return 0

        lax.fori_loop(0, CH, it_pair, 0, unroll=2)

        cp5 = pltpu.async_copy(x_hbm.at[idx_v.at[6, c]], e1_rows, sem0)
        cp6 = pltpu.async_copy(x_hbm.at[idx_v.at[7, c]], e2_rows, sem1)
        cp5.wait()
        cp6.wait()

        def us_pair(p, _):
            acc = jnp.zeros((LANES,), jnp.float32)
            for d in range(D // LANES):
                ds_ = pl.ds(d * LANES, LANES)
                xu = u_rows[p, ds_]
                dp = xu - e1_rows[p, ds_]
                dn = xu - e2_rows[p, ds_]
                acc += dp * dp - dn * dn
            us_b[off + p, :] = acc
            return 0

        lax.fori_loop(0, CH, us_pair, 0, unroll=2)
        return 0

    lax.fori_loop(0, NCHUNK, chunk, 0)

    pltpu.sync_copy(cf_b, cf_hbm.at[pl.ds(base, BPW)])
    pltpu.sync_copy(it_b, it_hbm.at[pl.ds(base, BPW)])
    pltpu.sync_copy(us_b, us_hbm.at[pl.ds(base, BPW)])


def _sc_partials(x, pnp_t):
    mesh = plsc.VectorSubcoreMesh(core_axis_name="c", subcore_axis_name="s")
    f32 = jnp.float32
    out = jax.ShapeDtypeStruct((B, LANES), f32)
    return pl.kernel(
        _sc_body,
        out_type=(out, out, out),
        mesh=mesh,
        compiler_params=pltpu.CompilerParams(use_tc_tiling_on_sc=False),
        scratch_types=[
            pltpu.VMEM((9, NCHUNK, CH), jnp.int32),
            pltpu.VMEM((CH, D), f32),
            pltpu.VMEM((CH, D), f32),
            pltpu.VMEM((CH, D), f32),
            pltpu.VMEM((CH, D), f32),
            pltpu.VMEM((BPW, LANES), f32),
            pltpu.VMEM((BPW, LANES), f32),
            pltpu.VMEM((BPW, LANES), f32),
            pltpu.SemaphoreType.DMA,
            pltpu.SemaphoreType.DMA,
            pltpu.SemaphoreType.DMA,
        ],
    )(x, pnp_t)


BLK_N = 4000


def _log_sigmoid(z):
    return jnp.minimum(z, 0.0) - jnp.log1p(jnp.exp(-jnp.abs(z)))


def _tc_body(x_ref, xm_ref, cf_ref, it_ref, us_ref, mi_ref, mu_ref, out_ref):
    i = pl.program_id(0)

    @pl.when(i == 0)
    def _():
        cf = jnp.sum(cf_ref[...], axis=1)
        it = jnp.sum(it_ref[...], axis=1) * mi_ref[...]
        us = jnp.sum(us_ref[...], axis=1) * mu_ref[...]
        cf_loss = -jnp.sum(_log_sigmoid(cf))
        reg_loss = -jnp.sum(_log_sigmoid(it)) - jnp.sum(_log_sigmoid(us))
        out_ref[0, 0] = cf_loss + ENTITY_AWARE_COFF * reg_loss

    d = x_ref[...] - xm_ref[...]
    out_ref[0, 0] += (EWC_LAMBDA / 2.0) * 1e-05 * jnp.sum(d * d)


def _tc_finalize(x, xm, cf, it, us, mi, mu):
    grid = N // BLK_N
    full = lambda i: (0, 0)
    full1 = lambda i: (0,)
    return pl.pallas_call(
        _tc_body,
        grid=(grid,),
        in_specs=[
            pl.BlockSpec((BLK_N, D), lambda i: (i, 0)),
            pl.BlockSpec((BLK_N, D), lambda i: (i, 0)),
            pl.BlockSpec((B, LANES), full),
            pl.BlockSpec((B, LANES), full),
            pl.BlockSpec((B, LANES), full),
            pl.BlockSpec((B,), full1),
            pl.BlockSpec((B,), full1),
        ],
        out_specs=pl.BlockSpec(memory_space=pltpu.SMEM),
        out_shape=jax.ShapeDtypeStruct((1, 1), jnp.float32),
    )(x, xm, cf, it, us, mi, mu)


@jax.jit
def kernel(x, pos_neg_pair_t, x_estimated_mean):
    pnp_t = pos_neg_pair_t.T
    mi = pnp_t[5].astype(jnp.float32)
    mu = pnp_t[8].astype(jnp.float32)
    pnp_g = pnp_t.reshape(9, NW, NCHUNK, CH)
    cf, it, us = _sc_partials(x, pnp_g)
    loss = _tc_finalize(x, x_estimated_mean, cf, it, us, mi, mu)
    return loss[0, 0]


# speedup vs baseline: 1.1935x; 1.1935x over previous
import jax
import jax.numpy as jnp
from jax import lax
from jax.experimental import pallas as pl
from jax.experimental.pallas import tpu as pltpu
from jax.experimental.pallas import tpu_sc as plsc

N = 100000
D = 64
B = 16384
ENTITY_AWARE_COFF = 0.001
EWC_LAMBDA = 100.0

NC = 2
NS = 16
LANES = 16
NW = NC * NS
BPW = B // NW
CH = 128
NCHUNK = BPW // CH
NG = CH // LANES


def _sc_body(x_hbm, pnp_hbm, cf_hbm, it_hbm, us_hbm,
             pnp_v, idx_v, u_rows, p_rows, e1_rows, e2_rows,
             stage, cf_b, it_b, us_b, sem0, sem1, sem2):
    wid = lax.axis_index("s") * NC + lax.axis_index("c")
    base = wid * BPW

    pltpu.sync_copy(pnp_hbm.at[pl.ds(base, BPW)], pnp_v)
    lane = lax.iota(jnp.int32, LANES)

    def extract(g, _):
        c = g // NG
        o = (g % NG) * LANES
        rows = g * LANES + lane
        for j in range(9):
            colv = jnp.full((LANES,), j, jnp.int32)
            idx_v[j, c, pl.ds(o, LANES)] = plsc.load_gather(pnp_v, [rows, colv])
        return 0

    lax.fori_loop(0, BPW // LANES, extract, 0)

    def flush_group(rows_fn, c, o, mask_row, out_b):
        for jj in range(LANES):
            acc = rows_fn(o * LANES + jj)
            plsc.store_scatter(stage, [lane, jnp.full((LANES,), jj, jnp.int32)],
                               acc)
        tot = stage[0, :]
        for l in range(1, LANES):
            tot = tot + stage[l, :]
        if mask_row is not None:
            m = idx_v[mask_row, c, pl.ds(o * LANES, LANES)].astype(jnp.float32)
            tot = tot * m
        out_b[pl.ds(c * CH + o * LANES, LANES)] = tot

    def chunk(c, _):
        cp0 = pltpu.async_copy(x_hbm.at[idx_v.at[0, c]], u_rows, sem0)
        cp1 = pltpu.async_copy(x_hbm.at[idx_v.at[1, c]], p_rows, sem1)
        cp2 = pltpu.async_copy(x_hbm.at[idx_v.at[2, c]], e1_rows, sem2)
        cp0.wait()
        cp1.wait()
        cp2.wait()

        def cf_rows(p):
            acc = jnp.zeros((LANES,), jnp.float32)
            for d in range(D // LANES):
                ds_ = pl.ds(d * LANES, LANES)
                acc += u_rows[p, ds_] * (p_rows[p, ds_] - e1_rows[p, ds_])
            return acc

        def cf_group(o, _):
            flush_group(cf_rows, c, o, None, cf_b)
            return 0

        lax.fori_loop(0, NG, cf_group, 0)

        cp3 = pltpu.async_copy(x_hbm.at[idx_v.at[3, c]], e1_rows, sem0)
        cp4 = pltpu.async_copy(x_hbm.at[idx_v.at[4, c]], e2_rows, sem1)
        cp3.wait()
        cp4.wait()

        def it_rows(p):
            acc = jnp.zeros((LANES,), jnp.float32)
            for d in range(D // LANES):
                ds_ = pl.ds(d * LANES, LANES)
                xi = p_rows[p, ds_]
                dp = xi - e1_rows[p, ds_]
                dn = xi - e2_rows[p, ds_]
                acc += dp * dp - dn * dn
            return acc

        def it_group(o, _):
            flush_group(it_rows, c, o, 5, it_b)
            return 0

        lax.fori_loop(0, NG, it_group, 0)

        cp5 = pltpu.async_copy(x_hbm.at[idx_v.at[6, c]], e1_rows, sem0)
        cp6 = pltpu.async_copy(x_hbm.at[idx_v.at[7, c]], e2_rows, sem1)
        cp5.wait()
        cp6.wait()

        def us_rows(p):
            acc = jnp.zeros((LANES,), jnp.float32)
            for d in range(D // LANES):
                ds_ = pl.ds(d * LANES, LANES)
                xu = u_rows[p, ds_]
                dp = xu - e1_rows[p, ds_]
                dn = xu - e2_rows[p, ds_]
                acc += dp * dp - dn * dn
            return acc

        def us_group(o, _):
            flush_group(us_rows, c, o, 8, us_b)
            return 0

        lax.fori_loop(0, NG, us_group, 0)
        return 0

    lax.fori_loop(0, NCHUNK, chunk, 0)

    pltpu.sync_copy(cf_b, cf_hbm.at[pl.ds(base, BPW)])
    pltpu.sync_copy(it_b, it_hbm.at[pl.ds(base, BPW)])
    pltpu.sync_copy(us_b, us_hbm.at[pl.ds(base, BPW)])


def _sc_partials(x, pnp):
    mesh = plsc.VectorSubcoreMesh(core_axis_name="c", subcore_axis_name="s")
    f32 = jnp.float32
    out = jax.ShapeDtypeStruct((B,), f32)
    return pl.kernel(
        _sc_body,
        out_type=(out, out, out),
        mesh=mesh,
        compiler_params=pltpu.CompilerParams(use_tc_tiling_on_sc=False,
                                             needs_layout_passes=False),
        scratch_types=[
            pltpu.VMEM((BPW, 9), jnp.int32),
            pltpu.VMEM((9, NCHUNK, CH), jnp.int32),
            pltpu.VMEM((CH, D), f32),
            pltpu.VMEM((CH, D), f32),
            pltpu.VMEM((CH, D), f32),
            pltpu.VMEM((CH, D), f32),
            pltpu.VMEM((LANES, LANES), f32),
            pltpu.VMEM((BPW,), f32),
            pltpu.VMEM((BPW,), f32),
            pltpu.VMEM((BPW,), f32),
            pltpu.SemaphoreType.DMA,
            pltpu.SemaphoreType.DMA,
            pltpu.SemaphoreType.DMA,
        ],
    )(x, pnp)


BLK_N = 4000


def _ewc_body(x_ref, xm_ref, out_ref):
    i = pl.program_id(0)

    @pl.when(i == 0)
    def _():
        out_ref[0, 0] = 0.0

    d = x_ref[...] - xm_ref[...]
    out_ref[0, 0] += (EWC_LAMBDA / 2.0) * 1e-05 * jnp.sum(d * d)


def _ewc(x, xm):
    return pl.pallas_call(
        _ewc_body,
        grid=(N // BLK_N,),
        in_specs=[
            pl.BlockSpec((BLK_N, D), lambda i: (i, 0)),
            pl.BlockSpec((BLK_N, D), lambda i: (i, 0)),
        ],
        out_specs=pl.BlockSpec(memory_space=pltpu.SMEM),
        out_shape=jax.ShapeDtypeStruct((1, 1), jnp.float32),
    )(x, xm)


def _log_sigmoid(z):
    return jnp.minimum(z, 0.0) - jnp.log1p(jnp.exp(-jnp.abs(z)))


def _final_body(cf_ref, it_ref, us_ref, ewc_ref, out_ref):
    cf_loss = -jnp.sum(_log_sigmoid(cf_ref[...]))
    reg_loss = (-jnp.sum(_log_sigmoid(it_ref[...]))
                - jnp.sum(_log_sigmoid(us_ref[...])))
    out_ref[0, 0] = cf_loss + ENTITY_AWARE_COFF * reg_loss + ewc_ref[0, 0]


def _finalize(cf, it, us, ewc):
    return pl.pallas_call(
        _final_body,
        in_specs=[
            pl.BlockSpec((B,), lambda: (0,)),
            pl.BlockSpec((B,), lambda: (0,)),
            pl.BlockSpec((B,), lambda: (0,)),
            pl.BlockSpec(memory_space=pltpu.SMEM),
        ],
        out_specs=pl.BlockSpec(memory_space=pltpu.SMEM),
        out_shape=jax.ShapeDtypeStruct((1, 1), jnp.float32),
    )(cf, it, us, ewc)


@jax.jit
def kernel(x, pos_neg_pair_t, x_estimated_mean):
    cf, it, us = _sc_partials(x, pos_neg_pair_t)
    ewc = _ewc(x, x_estimated_mean)
    loss = _finalize(cf, it, us, ewc)
    return loss[0, 0]


# speedup vs baseline: 1.6667x; 1.3964x over previous
import jax
import jax.numpy as jnp
from jax import lax
from jax.experimental import pallas as pl
from jax.experimental.pallas import tpu as pltpu
from jax.experimental.pallas import tpu_sc as plsc

N = 100000
D = 64
B = 16384
ENTITY_AWARE_COFF = 0.001
EWC_LAMBDA = 100.0

NC = 2
NS = 16
LANES = 16
NW = NC * NS
BPW = B // NW
CH = 128
NCHUNK = BPW // CH
NG = CH // LANES


def _sc_body(x_hbm, pnp_hbm, cf_hbm, it_hbm, us_hbm,
             pnp_v, idx_v, u_rows, p_rows, e1_rows, e2_rows,
             stage, cf_b, it_b, us_b, sem0, sem1, sem2):
    wid = lax.axis_index("s") * NC + lax.axis_index("c")
    base = wid * BPW

    pltpu.sync_copy(pnp_hbm.at[pl.ds(base, BPW)], pnp_v)
    lane = lax.iota(jnp.int32, LANES)

    def extract(g, _):
        c = g // NG
        o = (g % NG) * LANES
        rows = g * LANES + lane
        for j in range(9):
            colv = jnp.full((LANES,), j, jnp.int32)
            v = plsc.load_gather(pnp_v, [rows, colv])
            if j not in (5, 8):
                v = ((v >> 11) << 11) | ((v & 1023) << 1) | ((v >> 10) & 1)
            idx_v[j, c, pl.ds(o, LANES)] = v
        return 0

    lax.fori_loop(0, BPW // LANES, extract, 0)

    def flush_group(rows_fn, c, o, mask_row, out_b):
        for jj in range(LANES):
            acc = rows_fn(o * LANES + jj)
            plsc.store_scatter(stage, [lane, jnp.full((LANES,), jj, jnp.int32)],
                               acc)
        tot = stage[0, :]
        for l in range(1, LANES):
            tot = tot + stage[l, :]
        if mask_row is not None:
            m = idx_v[mask_row, c, pl.ds(o * LANES, LANES)].astype(jnp.float32)
            tot = tot * m
        out_b[pl.ds(c * CH + o * LANES, LANES)] = tot

    def chunk(c, _):
        cp0 = pltpu.async_copy(x_hbm.at[idx_v.at[0, c]], u_rows, sem0)
        cp1 = pltpu.async_copy(x_hbm.at[idx_v.at[1, c]], p_rows, sem1)
        cp2 = pltpu.async_copy(x_hbm.at[idx_v.at[2, c]], e1_rows, sem2)
        cp0.wait()
        cp1.wait()
        cp2.wait()

        def cf_rows(p):
            acc = jnp.zeros((LANES,), jnp.float32)
            for d in range(D // LANES):
                ds_ = pl.ds(d * LANES, LANES)
                acc += u_rows[p, ds_] * (p_rows[p, ds_] - e1_rows[p, ds_])
            return acc

        def cf_group(o, _):
            flush_group(cf_rows, c, o, None, cf_b)
            return 0

        lax.fori_loop(0, NG, cf_group, 0)

        cp3 = pltpu.async_copy(x_hbm.at[idx_v.at[3, c]], e1_rows, sem0)
        cp4 = pltpu.async_copy(x_hbm.at[idx_v.at[4, c]], e2_rows, sem1)
        cp3.wait()
        cp4.wait()

        def it_rows(p):
            acc = jnp.zeros((LANES,), jnp.float32)
            for d in range(D // LANES):
                ds_ = pl.ds(d * LANES, LANES)
                xi = p_rows[p, ds_]
                dp = xi - e1_rows[p, ds_]
                dn = xi - e2_rows[p, ds_]
                acc += dp * dp - dn * dn
            return acc

        def it_group(o, _):
            flush_group(it_rows, c, o, 5, it_b)
            return 0

        lax.fori_loop(0, NG, it_group, 0)

        cp5 = pltpu.async_copy(x_hbm.at[idx_v.at[6, c]], e1_rows, sem0)
        cp6 = pltpu.async_copy(x_hbm.at[idx_v.at[7, c]], e2_rows, sem1)
        cp5.wait()
        cp6.wait()

        def us_rows(p):
            acc = jnp.zeros((LANES,), jnp.float32)
            for d in range(D // LANES):
                ds_ = pl.ds(d * LANES, LANES)
                xu = u_rows[p, ds_]
                dp = xu - e1_rows[p, ds_]
                dn = xu - e2_rows[p, ds_]
                acc += dp * dp - dn * dn
            return acc

        def us_group(o, _):
            flush_group(us_rows, c, o, 8, us_b)
            return 0

        lax.fori_loop(0, NG, us_group, 0)
        return 0

    lax.fori_loop(0, NCHUNK, chunk, 0)

    pltpu.sync_copy(cf_b, cf_hbm.at[pl.ds(base, BPW)])
    pltpu.sync_copy(it_b, it_hbm.at[pl.ds(base, BPW)])
    pltpu.sync_copy(us_b, us_hbm.at[pl.ds(base, BPW)])


def _sc_partials(x, pnp):
    mesh = plsc.VectorSubcoreMesh(core_axis_name="c", subcore_axis_name="s")
    f32 = jnp.float32
    out = jax.ShapeDtypeStruct((B,), f32)
    return pl.kernel(
        _sc_body,
        out_type=(out, out, out),
        mesh=mesh,
        compiler_params=pltpu.CompilerParams(use_tc_tiling_on_sc=False,
                                             needs_layout_passes=False),
        scratch_types=[
            pltpu.VMEM((BPW, 9), jnp.int32),
            pltpu.VMEM((9, NCHUNK, CH), jnp.int32),
            pltpu.VMEM((CH, D), f32),
            pltpu.VMEM((CH, D), f32),
            pltpu.VMEM((CH, D), f32),
            pltpu.VMEM((CH, D), f32),
            pltpu.VMEM((LANES, LANES), f32),
            pltpu.VMEM((BPW,), f32),
            pltpu.VMEM((BPW,), f32),
            pltpu.VMEM((BPW,), f32),
            pltpu.SemaphoreType.DMA,
            pltpu.SemaphoreType.DMA,
            pltpu.SemaphoreType.DMA,
        ],
    )(x, pnp)


BLK_M = 2048
GRID_M = (N + BLK_M - 1) // BLK_M
N2 = GRID_M * BLK_M


def _prep_body(xt_ref, xmt_ref, xrm_ref, ewc_ref):
    i = pl.program_id(0)

    @pl.when(i == 0)
    def _():
        ewc_ref[0, 0] = 0.0

    xt = xt_ref[...]
    d = xt - xmt_ref[...]
    col = lax.broadcasted_iota(jnp.int32, (D, BLK_M), 1)
    d = jnp.where(col < N - i * BLK_M, d, 0.0)
    ewc_ref[0, 0] += (EWC_LAMBDA / 2.0) * 1e-05 * jnp.sum(d * d)
    t = jnp.transpose(xt)
    xrm_ref[:, :D] = t[: BLK_M // 2]
    xrm_ref[:, D:] = t[BLK_M // 2:]


def _prep(xt, xmt):
    return pl.pallas_call(
        _prep_body,
        grid=(GRID_M,),
        in_specs=[
            pl.BlockSpec((D, BLK_M), lambda i: (0, i)),
            pl.BlockSpec((D, BLK_M), lambda i: (0, i)),
        ],
        out_specs=(
            pl.BlockSpec((BLK_M // 2, 2 * D), lambda i: (i, 0)),
            pl.BlockSpec(memory_space=pltpu.SMEM),
        ),
        out_shape=(
            jax.ShapeDtypeStruct((N2 // 2, 2 * D), jnp.float32),
            jax.ShapeDtypeStruct((1, 1), jnp.float32),
        ),
    )(xt, xmt)


def _log_sigmoid(z):
    return jnp.minimum(z, 0.0) - jnp.log1p(jnp.exp(-jnp.abs(z)))


def _final_body(cf_ref, it_ref, us_ref, ewc_ref, out_ref):
    cf_loss = -jnp.sum(_log_sigmoid(cf_ref[...]))
    reg_loss = (-jnp.sum(_log_sigmoid(it_ref[...]))
                - jnp.sum(_log_sigmoid(us_ref[...])))
    out_ref[0, 0] = cf_loss + ENTITY_AWARE_COFF * reg_loss + ewc_ref[0, 0]


def _finalize(cf, it, us, ewc):
    return pl.pallas_call(
        _final_body,
        in_specs=[
            pl.BlockSpec((B,), lambda: (0,)),
            pl.BlockSpec((B,), lambda: (0,)),
            pl.BlockSpec((B,), lambda: (0,)),
            pl.BlockSpec(memory_space=pltpu.SMEM),
        ],
        out_specs=pl.BlockSpec(memory_space=pltpu.SMEM),
        out_shape=jax.ShapeDtypeStruct((1, 1), jnp.float32),
    )(cf, it, us, ewc)


@jax.jit
def kernel(x, pos_neg_pair_t, x_estimated_mean):
    x_rm, ewc = _prep(x.T, x_estimated_mean.T)
    cf, it, us = _sc_partials(x_rm.reshape(N2, D), pos_neg_pair_t)
    loss = _finalize(cf, it, us, ewc)
    return loss[0, 0]


# speedup vs baseline: 1.9151x; 1.1491x over previous
import jax
import jax.numpy as jnp
from jax import lax
from jax.experimental import pallas as pl
from jax.experimental.pallas import tpu as pltpu
from jax.experimental.pallas import tpu_sc as plsc

N = 100000
D = 64
B = 16384
ENTITY_AWARE_COFF = 0.001
EWC_LAMBDA = 100.0

NC = 2
NS = 16
LANES = 16
NW = NC * NS
BPW = B // NW
CH = 128
NCHUNK = BPW // CH
NG = CH // LANES

BLK_M = 2048
GRID_M = (N + BLK_M - 1) // BLK_M
N2 = GRID_M * BLK_M


def _repack_body(xt_ref, xrm_ref):
    t = jnp.transpose(xt_ref[...])
    xrm_ref[:, :D] = t[: BLK_M // 2]
    xrm_ref[:, D:] = t[BLK_M // 2:]


def _repack(xt):
    return pl.pallas_call(
        _repack_body,
        grid=(GRID_M,),
        in_specs=[pl.BlockSpec((D, BLK_M), lambda i: (0, i))],
        out_specs=pl.BlockSpec((BLK_M // 2, 2 * D), lambda i: (i, 0)),
        out_shape=jax.ShapeDtypeStruct((N2 // 2, 2 * D), jnp.float32),
    )(xt)


def _ewc_body(xt_ref, xmt_ref, ewc_ref):
    i = pl.program_id(0)

    @pl.when(i == 0)
    def _():
        ewc_ref[0, 0] = 0.0

    d = xt_ref[...] - xmt_ref[...]
    col = lax.broadcasted_iota(jnp.int32, (D, BLK_M), 1)
    d = jnp.where(col < N - i * BLK_M, d, 0.0)
    ewc_ref[0, 0] += (EWC_LAMBDA / 2.0) * 1e-05 * jnp.sum(d * d)


def _ewc(xt, xmt):
    return pl.pallas_call(
        _ewc_body,
        grid=(GRID_M,),
        in_specs=[
            pl.BlockSpec((D, BLK_M), lambda i: (0, i)),
            pl.BlockSpec((D, BLK_M), lambda i: (0, i)),
        ],
        out_specs=pl.BlockSpec(memory_space=pltpu.SMEM),
        out_shape=jax.ShapeDtypeStruct((1, 1), jnp.float32),
    )(xt, xmt)


def _sc_body(x_hbm, i0, i1, i2, i3, i4, i5, i6, cf_hbm, it_hbm, us_hbm,
             idx_v, u_rows, p_rows, e1_rows, e2_rows,
             stage, cf_b, it_b, us_b, sem0, sem1, sem2):
    wid = lax.axis_index("s") * NC + lax.axis_index("c")
    base = wid * BPW
    lane = lax.iota(jnp.int32, LANES)

    for j, ih in enumerate((i0, i1, i2, i3, i4, i5, i6)):
        pltpu.sync_copy(ih.at[wid], idx_v.at[j])

    def flush_group(rows_fn, c, o, out_b):
        for jj in range(LANES):
            acc = rows_fn(o * LANES + jj)
            plsc.store_scatter(stage, [lane, jnp.full((LANES,), jj, jnp.int32)],
                               acc)
        tot = stage[0, :]
        for l in range(1, LANES):
            tot = tot + stage[l, :]
        out_b[pl.ds(c * CH + o * LANES, LANES)] = tot

    def chunk(c, _):
        cp0 = pltpu.async_copy(x_hbm.at[idx_v.at[0, c]], u_rows, sem0)
        cp1 = pltpu.async_copy(x_hbm.at[idx_v.at[1, c]], p_rows, sem1)
        cp2 = pltpu.async_copy(x_hbm.at[idx_v.at[2, c]], e1_rows, sem2)
        cp0.wait()
        cp1.wait()
        cp2.wait()

        def cf_rows(p):
            acc = jnp.zeros((LANES,), jnp.float32)
            for d in range(D // LANES):
                ds_ = pl.ds(d * LANES, LANES)
                acc += u_rows[p, ds_] * (p_rows[p, ds_] - e1_rows[p, ds_])
            return acc

        def cf_group(o, _):
            flush_group(cf_rows, c, o, cf_b)
            return 0

        lax.fori_loop(0, NG, cf_group, 0)

        cp3 = pltpu.async_copy(x_hbm.at[idx_v.at[3, c]], e1_rows, sem0)
        cp4 = pltpu.async_copy(x_hbm.at[idx_v.at[4, c]], e2_rows, sem1)
        cp3.wait()
        cp4.wait()

        def it_rows(p):
            acc = jnp.zeros((LANES,), jnp.float32)
            for d in range(D // LANES):
                ds_ = pl.ds(d * LANES, LANES)
                xi = p_rows[p, ds_]
                dp = xi - e1_rows[p, ds_]
                dn = xi - e2_rows[p, ds_]
                acc += dp * dp - dn * dn
            return acc

        def it_group(o, _):
            flush_group(it_rows, c, o, it_b)
            return 0

        lax.fori_loop(0, NG, it_group, 0)

        cp5 = pltpu.async_copy(x_hbm.at[idx_v.at[5, c]], e1_rows, sem0)
        cp6 = pltpu.async_copy(x_hbm.at[idx_v.at[6, c]], e2_rows, sem1)
        cp5.wait()
        cp6.wait()

        def us_rows(p):
            acc = jnp.zeros((LANES,), jnp.float32)
            for d in range(D // LANES):
                ds_ = pl.ds(d * LANES, LANES)
                xu = u_rows[p, ds_]
                dp = xu - e1_rows[p, ds_]
                dn = xu - e2_rows[p, ds_]
                acc += dp * dp - dn * dn
            return acc

        def us_group(o, _):
            flush_group(us_rows, c, o, us_b)
            return 0

        lax.fori_loop(0, NG, us_group, 0)
        return 0

    lax.fori_loop(0, NCHUNK, chunk, 0)

    pltpu.sync_copy(cf_b, cf_hbm.at[pl.ds(base, BPW)])
    pltpu.sync_copy(it_b, it_hbm.at[pl.ds(base, BPW)])
    pltpu.sync_copy(us_b, us_hbm.at[pl.ds(base, BPW)])


def _sc_partials(x_rm, idx_cols):
    mesh = plsc.VectorSubcoreMesh(core_axis_name="c", subcore_axis_name="s")
    f32 = jnp.float32
    out = jax.ShapeDtypeStruct((B,), f32)
    return pl.kernel(
        _sc_body,
        out_type=(out, out, out),
        mesh=mesh,
        compiler_params=pltpu.CompilerParams(use_tc_tiling_on_sc=False,
                                             needs_layout_passes=False),
        scratch_types=[
            pltpu.VMEM((7, NCHUNK, CH), jnp.int32),
            pltpu.VMEM((CH, D), f32),
            pltpu.VMEM((CH, D), f32),
            pltpu.VMEM((CH, D), f32),
            pltpu.VMEM((CH, D), f32),
            pltpu.VMEM((LANES, LANES), f32),
            pltpu.VMEM((BPW,), f32),
            pltpu.VMEM((BPW,), f32),
            pltpu.VMEM((BPW,), f32),
            pltpu.SemaphoreType.DMA,
            pltpu.SemaphoreType.DMA,
            pltpu.SemaphoreType.DMA,
        ],
    )(x_rm, *idx_cols)


def _log_sigmoid(z):
    return jnp.minimum(z, 0.0) - jnp.log1p(jnp.exp(-jnp.abs(z)))


def _final_body(cf_ref, it_ref, us_ref, mi_ref, mu_ref, ewc_ref, out_ref):
    cf_loss = -jnp.sum(_log_sigmoid(cf_ref[...]))
    reg_loss = (-jnp.sum(_log_sigmoid(it_ref[...] * mi_ref[...]))
                - jnp.sum(_log_sigmoid(us_ref[...] * mu_ref[...])))
    out_ref[0, 0] = cf_loss + ENTITY_AWARE_COFF * reg_loss + ewc_ref[0, 0]


def _finalize(cf, it, us, mi, mu, ewc):
    vec = pl.BlockSpec((B,), lambda: (0,))
    return pl.pallas_call(
        _final_body,
        in_specs=[vec, vec, vec, vec, vec,
                  pl.BlockSpec(memory_space=pltpu.SMEM)],
        out_specs=pl.BlockSpec(memory_space=pltpu.SMEM),
        out_shape=jax.ShapeDtypeStruct((1, 1), jnp.float32),
    )(cf, it, us, mi, mu, ewc)


@jax.jit
def kernel(x, pos_neg_pair_t, x_estimated_mean):
    idx_cols = []
    for j in (0, 1, 2, 3, 4, 6, 7):
        v = pos_neg_pair_t[:, j]
        v = ((v >> 11) << 11) | ((v & 1023) << 1) | ((v >> 10) & 1)
        idx_cols.append(v.reshape(NW, NCHUNK, CH))
    mi = pos_neg_pair_t[:, 5].astype(jnp.float32)
    mu = pos_neg_pair_t[:, 8].astype(jnp.float32)

    x_rm = _repack(x.T)
    ewc = _ewc(x.T, x_estimated_mean.T)
    cf, it, us = _sc_partials(x_rm.reshape(N2, D), idx_cols)
    loss = _finalize(cf, it, us, mi, mu, ewc)
    return loss[0, 0]


# speedup vs baseline: 2.1376x; 1.1161x over previous
import jax
import jax.numpy as jnp
from jax import lax
from jax.experimental import pallas as pl
from jax.experimental.pallas import tpu as pltpu
from jax.experimental.pallas import tpu_sc as plsc

N = 100000
D = 64
B = 16384
ENTITY_AWARE_COFF = 0.001
EWC_LAMBDA = 100.0

NC = 2
NS = 16
LANES = 16
NW = NC * NS
BPW = B // NW
CH = 128
NCHUNK = BPW // CH
NG = CH // LANES

BLK_M = 2048
GRID_M = (N + BLK_M - 1) // BLK_M
N2 = GRID_M * BLK_M


def _prep_body(xt_ref, xmt_ref, xrm_ref, ewc_ref):
    i = pl.program_id(0)

    @pl.when(i == 0)
    def _():
        ewc_ref[0, 0] = 0.0

    xt = xt_ref[...]
    d = xt - xmt_ref[...]
    col = lax.broadcasted_iota(jnp.int32, (D, BLK_M), 1)
    d = jnp.where(col < N - i * BLK_M, d, 0.0)
    ewc_ref[0, 0] += (EWC_LAMBDA / 2.0) * 1e-05 * jnp.sum(d * d)
    t = jnp.transpose(xt)
    xrm_ref[:, :D] = t[: BLK_M // 2]
    xrm_ref[:, D:] = t[BLK_M // 2:]


def _prep(xt, xmt):
    return pl.pallas_call(
        _prep_body,
        grid=(GRID_M,),
        in_specs=[
            pl.BlockSpec((D, BLK_M), lambda i: (0, i)),
            pl.BlockSpec((D, BLK_M), lambda i: (0, i)),
        ],
        out_specs=(
            pl.BlockSpec((BLK_M // 2, 2 * D), lambda i: (i, 0)),
            pl.BlockSpec(memory_space=pltpu.SMEM),
        ),
        out_shape=(
            jax.ShapeDtypeStruct((N2 // 2, 2 * D), jnp.float32),
            jax.ShapeDtypeStruct((1, 1), jnp.float32),
        ),
    )(xt, xmt)


def _sc_body(x_hbm, i0, i1, i2, i3, i4, i5, i6, cf_hbm, it_hbm, us_hbm,
             idx_v,
             b0_u, b0_p, b0_n, b0_e1, b0_e2, b0_e3, b0_e4,
             b1_u, b1_p, b1_n, b1_e1, b1_e2, b1_e3, b1_e4,
             st_cf, st_it, st_us, cf_b, it_b, us_b, *sems):
    wid = lax.axis_index("s") * NC + lax.axis_index("c")
    base = wid * BPW
    lane = lax.iota(jnp.int32, LANES)
    banks = ((b0_u, b0_p, b0_n, b0_e1, b0_e2, b0_e3, b0_e4),
             (b1_u, b1_p, b1_n, b1_e1, b1_e2, b1_e3, b1_e4))

    for j, ih in enumerate((i0, i1, i2, i3, i4, i5, i6)):
        pltpu.sync_copy(ih.at[wid], idx_v.at[j])

    def fire(c):
        b = banks[c % 2]
        s = sems[(c % 2) * 7:(c % 2) * 7 + 7]
        return [pltpu.async_copy(x_hbm.at[idx_v.at[j, c]], b[j], s[j])
                for j in range(7)]

    cps = fire(0)
    for c in range(NCHUNK):
        for cp in cps:
            cp.wait()
        cps = fire(c + 1) if c + 1 < NCHUNK else []
        u_r, p_r, n_r, e1_r, e2_r, e3_r, e4_r = banks[c % 2]

        def pair_step(p, _):
            cf = jnp.zeros((LANES,), jnp.float32)
            it = jnp.zeros((LANES,), jnp.float32)
            us = jnp.zeros((LANES,), jnp.float32)
            for d in range(D // LANES):
                ds_ = pl.ds(d * LANES, LANES)
                xu = u_r[p, ds_]
                xi = p_r[p, ds_]
                cf += xu * (xi - n_r[p, ds_])
                dip = xi - e1_r[p, ds_]
                din = xi - e2_r[p, ds_]
                it += dip * dip - din * din
                dup = xu - e3_r[p, ds_]
                dun = xu - e4_r[p, ds_]
                us += dup * dup - dun * dun
            jj = p & (LANES - 1)
            colv = jnp.full((LANES,), jj, jnp.int32)
            plsc.store_scatter(st_cf, [lane, colv], cf)
            plsc.store_scatter(st_it, [lane, colv], it)
            plsc.store_scatter(st_us, [lane, colv], us)

            @pl.when(jj == LANES - 1)
            def _():
                o = pl.ds(c * CH + p - (LANES - 1), LANES)
                for st, out_b in ((st_cf, cf_b), (st_it, it_b), (st_us, us_b)):
                    tot = st[0, :]
                    for l in range(1, LANES):
                        tot = tot + st[l, :]
                    out_b[o] = tot

            return 0

        lax.fori_loop(0, CH, pair_step, 0, unroll=2)

    pltpu.sync_copy(cf_b, cf_hbm.at[pl.ds(base, BPW)])
    pltpu.sync_copy(it_b, it_hbm.at[pl.ds(base, BPW)])
    pltpu.sync_copy(us_b, us_hbm.at[pl.ds(base, BPW)])


def _sc_partials(x_rm, idx_cols):
    mesh = plsc.VectorSubcoreMesh(core_axis_name="c", subcore_axis_name="s")
    f32 = jnp.float32
    out = jax.ShapeDtypeStruct((B,), f32)
    rows = pltpu.VMEM((CH, D), f32)
    stage = pltpu.VMEM((LANES, LANES), f32)
    outb = pltpu.VMEM((BPW,), f32)
    return pl.kernel(
        _sc_body,
        out_type=(out, out, out),
        mesh=mesh,
        compiler_params=pltpu.CompilerParams(use_tc_tiling_on_sc=False,
                                             needs_layout_passes=False),
        scratch_types=(
            [pltpu.VMEM((7, NCHUNK, CH), jnp.int32)]
            + [rows] * 14
            + [stage] * 3
            + [outb] * 3
            + [pltpu.SemaphoreType.DMA] * 14
        ),
    )(x_rm, *idx_cols)


def _log_sigmoid(z):
    return jnp.minimum(z, 0.0) - jnp.log1p(jnp.exp(-jnp.abs(z)))


def _final_body(cf_ref, it_ref, us_ref, mi_ref, mu_ref, ewc_ref, out_ref):
    cf_loss = -jnp.sum(_log_sigmoid(cf_ref[...]))
    reg_loss = (-jnp.sum(_log_sigmoid(it_ref[...] * mi_ref[...]))
                - jnp.sum(_log_sigmoid(us_ref[...] * mu_ref[...])))
    out_ref[0, 0] = cf_loss + ENTITY_AWARE_COFF * reg_loss + ewc_ref[0, 0]


def _finalize(cf, it, us, mi, mu, ewc):
    vec = pl.BlockSpec((B,), lambda: (0,))
    return pl.pallas_call(
        _final_body,
        in_specs=[vec, vec, vec, vec, vec,
                  pl.BlockSpec(memory_space=pltpu.SMEM)],
        out_specs=pl.BlockSpec(memory_space=pltpu.SMEM),
        out_shape=jax.ShapeDtypeStruct((1, 1), jnp.float32),
    )(cf, it, us, mi, mu, ewc)


@jax.jit
def kernel(x, pos_neg_pair_t, x_estimated_mean):
    idx_cols = []
    for j in (0, 1, 2, 3, 4, 6, 7):
        v = pos_neg_pair_t[:, j]
        v = ((v >> 11) << 11) | ((v & 1023) << 1) | ((v >> 10) & 1)
        idx_cols.append(v.reshape(NW, NCHUNK, CH))
    mi = pos_neg_pair_t[:, 5].astype(jnp.float32)
    mu = pos_neg_pair_t[:, 8].astype(jnp.float32)

    x_rm, ewc = _prep(x.T, x_estimated_mean.T)
    cf, it, us = _sc_partials(x_rm.reshape(N2, D), idx_cols)
    loss = _finalize(cf, it, us, mi, mu, ewc)
    return loss[0, 0]
